# Initial kernel scaffold; baseline (speedup 1.0000x reference)
import jax
import jax.numpy as jnp
from jax.experimental import pallas as pl


def kernel(indices, emb, W, b):
    raise NotImplementedError("write your pallas kernel here")


# speedup vs baseline: 73.2104x; 73.2104x over previous
import functools

import jax
import jax.numpy as jnp
from jax import lax
from jax.experimental import pallas as pl
from jax.experimental.pallas import tpu as pltpu
from jax.experimental.pallas import tpu_sc as plsc

_PAD_ROW = 1
_LANES = 16


def _table_body(emb_ref, w_ref, b_ref, out_ref):
    v = jnp.dot(emb_ref[...], w_ref[...], preferred_element_type=jnp.float32)
    row = lax.broadcasted_iota(jnp.int32, v.shape, 0)
    out_ref[...] = jnp.where(row == _PAD_ROW, 0.0, v) + b_ref[...]


def _make_table(emb, w, b):
    vocab = emb.shape[0]
    return pl.pallas_call(
        _table_body,
        out_shape=jax.ShapeDtypeStruct((vocab, 1), jnp.float32),
    )(emb, w, b.reshape(1, 1))


def _make_gather(vocab, batch, n_workers, b_per_w):
    mesh = plsc.VectorSubcoreMesh(core_axis_name="c", subcore_axis_name="s")

    @functools.partial(
        pl.kernel,
        mesh=mesh,
        out_type=jax.ShapeDtypeStruct((batch,), jnp.float32),
        scratch_types=[
            pltpu.VMEM((b_per_w,), jnp.int32),
            pltpu.VMEM((b_per_w,), jnp.float32),
            pltpu.SemaphoreType.DMA,
        ],
    )
    def gather_kernel(table_hbm, idx_hbm, out_hbm, idx_v, out_v, sem):
        wid = lax.axis_index("s") * 2 + lax.axis_index("c")
        base = wid * b_per_w
        pltpu.sync_copy(idx_hbm.at[pl.ds(base, b_per_w)], idx_v)
        pltpu.async_copy(table_hbm.at[idx_v], out_v, sem).wait()
        pltpu.sync_copy(out_v, out_hbm.at[pl.ds(base, b_per_w)])

    return gather_kernel


def kernel(indices, emb, W, b):
    idx0 = indices[0, :, 0].astype(jnp.int32)
    batch = idx0.shape[0]
    vocab = emb.shape[0]
    n_workers = 32
    b_per_w = batch // n_workers
    table = _make_table(emb, W, b).reshape(vocab)
    out = _make_gather(vocab, batch, n_workers, b_per_w)(table, idx0)
    return out.reshape(batch, 1)


# speedup vs baseline: 98.2789x; 1.3424x over previous
import functools

import jax
import jax.numpy as jnp
from jax import lax
from jax.experimental import pallas as pl
from jax.experimental.pallas import tpu as pltpu
from jax.experimental.pallas import tpu_sc as plsc

_PAD_ROW = 1
_LANES = 16


def _table_body(emb_ref, w_ref, b_ref, out_ref):
    v = jnp.dot(emb_ref[...], w_ref[...], preferred_element_type=jnp.float32)
    row = lax.broadcasted_iota(jnp.int32, v.shape, 0)
    out_ref[...] = jnp.where(row == _PAD_ROW, 0.0, v) + b_ref[...]


def _make_table(emb, w, b):
    vocab = emb.shape[0]
    return pl.pallas_call(
        _table_body,
        out_shape=jax.ShapeDtypeStruct((vocab, 1), jnp.float32),
    )(emb, w, b.reshape(1, 1))


def _make_gather(vocab, batch, n_workers, b_per_w):
    mesh = plsc.VectorSubcoreMesh(core_axis_name="c", subcore_axis_name="s")

    @functools.partial(
        pl.kernel,
        mesh=mesh,
        out_type=jax.ShapeDtypeStruct((batch,), jnp.float32),
        scratch_types=[
            pltpu.VMEM((b_per_w,), jnp.int32),
            pltpu.VMEM((b_per_w,), jnp.float32),
            pltpu.VMEM_SHARED((vocab,), jnp.float32),
            pltpu.SemaphoreType.DMA,
        ],
    )
    def gather_kernel(table_hbm, idx_hbm, out_hbm, idx_v, out_v, tab_s, sem):
        sid = lax.axis_index("s")
        wid = sid * 2 + lax.axis_index("c")
        base = wid * b_per_w
        pltpu.sync_copy(idx_hbm.at[pl.ds(base, b_per_w)], idx_v)
        @pl.when(sid == 0)
        def _():
            pltpu.sync_copy(table_hbm, tab_s)
        plsc.subcore_barrier()
        pltpu.async_copy(tab_s.at[idx_v], out_v, sem).wait()
        pltpu.sync_copy(out_v, out_hbm.at[pl.ds(base, b_per_w)])

    return gather_kernel


def kernel(indices, emb, W, b):
    idx0 = indices[0, :, 0].astype(jnp.int32)
    batch = idx0.shape[0]
    vocab = emb.shape[0]
    n_workers = 32
    b_per_w = batch // n_workers
    table = _make_table(emb, W, b).reshape(vocab)
    out = _make_gather(vocab, batch, n_workers, b_per_w)(table, idx0)
    return out.reshape(batch, 1)
